# baseline (speedup 1.0000x reference)
import functools

import jax
import jax.numpy as jnp
from jax import lax
from jax.experimental import pallas as pl
from jax.experimental.pallas import tpu as pltpu
from jax.experimental.pallas import tpu_sc as plsc

F32 = jnp.float32
I32 = jnp.int32

_SC_PARAMS = pltpu.CompilerParams(use_tc_tiling_on_sc=False,
                                  needs_layout_passes=False)

NC = 2
NS = 16
NW = NC * NS
L = 16

D = 16
GB = 64


def _cdiv(a, b):
    return (a + b - 1) // b


def _encode_body(emb_ref, w1_ref, b1_ref, g_ref):
    acc = lax.dot_general(emb_ref[...], w1_ref[...],
                          (((1,), (0,)), ((), ())),
                          preferred_element_type=F32)
    g_ref[...] = jnp.maximum(acc + b1_ref[...], 0.0)


def _encode(emb, W1, b1):
    n, dfeat = emb.shape
    blk = 2000
    return pl.pallas_call(
        _encode_body,
        grid=(n // blk,),
        in_specs=[
            pl.BlockSpec((blk, dfeat), lambda i: (i, 0)),
            pl.BlockSpec((dfeat, D), lambda i: (0, 0)),
            pl.BlockSpec((1, D), lambda i: (0, 0)),
        ],
        out_specs=pl.BlockSpec((blk, D), lambda i: (i, 0)),
        out_shape=jax.ShapeDtypeStruct((n, D), F32),
    )(emb, W1, b1.reshape(1, D))


def _prep_body(beta_ref, h_ref, t32_ref, tn_ref):
    h = h_ref[...]
    beta = beta_ref[0]
    nrm = jnp.sqrt(jnp.sum(h * h, axis=-1, keepdims=True))
    xn = h / jnp.maximum(nrm, 1e-12)
    t32_ref[...] = jnp.concatenate([beta * xn, h], axis=-1)
    tn_ref[...] = xn


def _prep_tables(h, beta, np_):
    blk = 1024
    return pl.pallas_call(
        _prep_body,
        grid=(np_ // blk,),
        in_specs=[
            pl.BlockSpec(memory_space=pltpu.SMEM),
            pl.BlockSpec((blk, D), lambda i: (i, 0)),
        ],
        out_specs=[
            pl.BlockSpec((blk, 2 * D), lambda i: (i, 0)),
            pl.BlockSpec((blk, D), lambda i: (i, 0)),
        ],
        out_shape=[
            jax.ShapeDtypeStruct((np_, 2 * D), F32),
            jax.ShapeDtypeStruct((np_, D), F32),
        ],
    )(jnp.reshape(beta.astype(F32), (1,)), h)


def _agnn_out(h, acc, den, beta):
    nrm2 = jnp.sum(h * h, axis=-1, keepdims=True)
    nrm = jnp.sqrt(nrm2)
    inv = 1.0 / jnp.maximum(nrm, 1e-12)
    s = nrm2 * inv * inv
    ex_self = jnp.exp(beta * s)
    num = acc[0] + acc[1] + ex_self * h
    dfull = (den[0] + den[1])[:, None] + ex_self
    return num / jnp.maximum(dfull, 1e-16)


def _fin_prep_body(betas_ref, h_ref, acc_ref, den_ref, h1_ref, t32_ref,
                   tn_ref):
    h1 = _agnn_out(h_ref[...], acc_ref[...], den_ref[...], betas_ref[0])
    h1_ref[...] = h1
    nrm = jnp.sqrt(jnp.sum(h1 * h1, axis=-1, keepdims=True))
    xn = h1 / jnp.maximum(nrm, 1e-12)
    t32_ref[...] = jnp.concatenate([betas_ref[1] * xn, h1], axis=-1)
    tn_ref[...] = xn


def _fin_prep(h, acc, den, beta, beta_next, np_):
    blk = 1024
    betas = jnp.stack([beta.astype(F32), beta_next.astype(F32)])
    return pl.pallas_call(
        _fin_prep_body,
        grid=(np_ // blk,),
        in_specs=[
            pl.BlockSpec(memory_space=pltpu.SMEM),
            pl.BlockSpec((blk, D), lambda i: (i, 0)),
            pl.BlockSpec((2, blk, D), lambda i: (0, i, 0)),
            pl.BlockSpec((2, blk), lambda i: (0, i)),
        ],
        out_specs=[
            pl.BlockSpec((blk, D), lambda i: (i, 0)),
            pl.BlockSpec((blk, 2 * D), lambda i: (i, 0)),
            pl.BlockSpec((blk, D), lambda i: (i, 0)),
        ],
        out_shape=[
            jax.ShapeDtypeStruct((np_, D), F32),
            jax.ShapeDtypeStruct((np_, 2 * D), F32),
            jax.ShapeDtypeStruct((np_, D), F32),
        ],
    )(betas, h, acc, den)


def _fin_head_body(beta_ref, h_ref, acc_ref, den_ref, w2_ref, b2_ref,
                   out_ref):
    h2 = _agnn_out(h_ref[...], acc_ref[...], den_ref[...], beta_ref[0])
    logits = lax.dot_general(h2, w2_ref[...],
                             (((1,), (0,)), ((), ())),
                             preferred_element_type=F32) + b2_ref[...]
    m = jnp.max(logits, axis=-1, keepdims=True)
    lse = jnp.log(jnp.sum(jnp.exp(logits - m), axis=-1, keepdims=True)) + m
    out_ref[...] = logits - lse


def _fin_head(h, acc, den, beta, W2, b2, np_):
    blk = 1024
    return pl.pallas_call(
        _fin_head_body,
        grid=(np_ // blk,),
        in_specs=[
            pl.BlockSpec(memory_space=pltpu.SMEM),
            pl.BlockSpec((blk, D), lambda i: (i, 0)),
            pl.BlockSpec((2, blk, D), lambda i: (0, i, 0)),
            pl.BlockSpec((2, blk), lambda i: (0, i)),
            pl.BlockSpec((D, D), lambda i: (0, 0)),
            pl.BlockSpec((1, D), lambda i: (0, 0)),
        ],
        out_specs=pl.BlockSpec((blk, D), lambda i: (i, 0)),
        out_shape=jax.ShapeDtypeStruct((np_, D), F32),
    )(jnp.reshape(beta.astype(F32), (1,)), h, acc, den, W2,
      b2.reshape(1, D))


def _gather_rows(table, idx, np_):
    n_rows = np_ // NW
    n_grp = n_rows // GB
    mesh = plsc.VectorSubcoreMesh(core_axis_name="c", subcore_axis_name="s")

    @functools.partial(
        pl.kernel,
        out_type=jax.ShapeDtypeStruct((np_, D), F32),
        mesh=mesh,
        compiler_params=_SC_PARAMS,
        scratch_types=[
            pltpu.VMEM((GB,), I32),
            pltpu.VMEM((GB, D), F32),
            pltpu.SemaphoreType.DMA,
        ],
    )
    def gk(table_hbm, idx_hbm, out_hbm, idx_v, rows_v, sem):
        wid = lax.axis_index("s") * NC + lax.axis_index("c")
        base = wid * n_rows

        @pl.loop(0, n_grp)
        def _grp(i):
            off = base + i * GB
            pltpu.sync_copy(idx_hbm.at[pl.ds(off, GB)], idx_v)
            pltpu.async_copy(table_hbm.at[idx_v], rows_v, sem).wait()
            pltpu.sync_copy(rows_v, out_hbm.at[pl.ds(off, GB)])

    return gk(table, idx)


CH = 16
NB = 4


def _edge_pass(t32, tn, src2d, dst2d, zer_acc, zer_den, nsp, epw):
    n_grp = epw // GB
    rows_per_sub = nsp // NS
    mesh = plsc.VectorSubcoreMesh(core_axis_name="c", subcore_axis_name="s")

    @functools.partial(
        pl.kernel,
        out_type=[
            jax.ShapeDtypeStruct((NC, nsp, D), F32),
            jax.ShapeDtypeStruct((NC, nsp), F32),
        ],
        mesh=mesh,
        compiler_params=_SC_PARAMS,
        scratch_types=(
            [pltpu.VMEM((2 * CH, GB), I32)] * 2
            + [pltpu.VMEM((GB, 2 * D), F32)] * NB
            + [pltpu.VMEM((GB, D), F32)] * NB
            + [pltpu.VMEM((GB, D), F32)] * NB
            + [pltpu.VMEM((GB,), F32)] * NB
            + [pltpu.VMEM_SHARED((nsp, D), F32),
               pltpu.VMEM_SHARED((nsp,), F32)]
            + [pltpu.SemaphoreType.DMA] * (2 * NB)
        ),
    )
    def ek(t32_hbm, tn_hbm, src_hbm, dst_hbm, za_hbm, zd_hbm,
           acc_out, den_out, sidx, didx, *rest):
        srows = rest[0:NB]
        drows = rest[NB:2 * NB]
        pbuf = rest[2 * NB:3 * NB]
        exbuf = rest[3 * NB:4 * NB]
        acc_sh = rest[4 * NB]
        den_sh = rest[4 * NB + 1]
        semg = rest[4 * NB + 2:5 * NB + 2]
        sems = rest[5 * NB + 2:6 * NB + 2]
        cid = lax.axis_index("c")
        sid = lax.axis_index("s")
        wid = sid * NC + cid

        zoff = sid * rows_per_sub
        pltpu.sync_copy(za_hbm.at[pl.ds(zoff, rows_per_sub)],
                        acc_sh.at[pl.ds(zoff, rows_per_sub)])
        pltpu.sync_copy(zd_hbm.at[pl.ds(zoff, rows_per_sub)],
                        den_sh.at[pl.ds(zoff, rows_per_sub)])
        plsc.subcore_barrier()

        gbase = wid * n_grp
        lanes = jnp.arange(L, dtype=I32)

        def stage(c):
            par = (c % 2) * CH
            pltpu.sync_copy(src_hbm.at[pl.ds(gbase + c * CH, CH)],
                            sidx.at[pl.ds(par, CH)])
            pltpu.sync_copy(dst_hbm.at[pl.ds(gbase + c * CH, CH)],
                            didx.at[pl.ds(par, CH)])

        def idxrow(j):
            return (j % CH) + ((j // CH) % 2) * CH

        def start_gather(b, j):
            row = idxrow(j)
            pltpu.async_copy(t32_hbm.at[sidx.at[row]], srows[b], semg[b])
            pltpu.async_copy(tn_hbm.at[didx.at[row]], drows[b], semg[b])

        def wait_gather(b):
            pltpu.make_async_copy(t32_hbm.at[sidx.at[0]], srows[b],
                                  semg[b]).wait()
            pltpu.make_async_copy(tn_hbm.at[didx.at[0]], drows[b],
                                  semg[b]).wait()

        def start_scatter(b, j):
            row = idxrow(j)
            pltpu.async_copy(pbuf[b], acc_sh.at[didx.at[row]], sems[b],
                             add=True)
            pltpu.async_copy(exbuf[b], den_sh.at[didx.at[row]], sems[b],
                             add=True)

        def wait_scatter(b):
            pltpu.make_async_copy(pbuf[b], acc_sh.at[didx.at[0]],
                                  sems[b]).wait()
            pltpu.make_async_copy(exbuf[b], den_sh.at[didx.at[0]],
                                  sems[b]).wait()

        def compute(b):
            @pl.loop(0, GB // L)
            def _blk(eb):
                eidx = eb * L + lanes
                acc = jnp.zeros((L,), F32)
                for f in range(D):
                    fs = jnp.full((L,), f, I32)
                    a = plsc.load_gather(srows[b], [eidx, fs])
                    bb = plsc.load_gather(drows[b], [eidx, fs])
                    acc = acc + a * bb
                ex = jnp.exp(acc)
                plsc.store_scatter(exbuf[b], [eidx], ex)
                for f in range(D):
                    fs = jnp.full((L,), f, I32)
                    xv = plsc.load_gather(srows[b],
                                          [eidx, jnp.full((L,), D + f, I32)])
                    plsc.store_scatter(pbuf[b], [eidx, fs], ex * xv)

        stage(0)
        for b in range(NB):
            start_gather(b, b)

        @pl.loop(0, n_grp // NB)
        def _round(q):
            for b in range(NB):
                j = q * NB + b
                wait_gather(b)

                @pl.when(q > 0)
                def _():
                    wait_scatter(b)

                compute(b)
                start_scatter(b, j)
                jn = j + NB

                @pl.when(jn < n_grp)
                def _():
                    @pl.when(jn % CH == 0)
                    def _():
                        stage(jn // CH)

                    start_gather(b, jn)

        for b in range(NB):
            wait_scatter(b)
        plsc.subcore_barrier()
        pltpu.sync_copy(acc_sh.at[pl.ds(zoff, rows_per_sub)],
                        acc_out.at[cid, pl.ds(zoff, rows_per_sub)])
        pltpu.sync_copy(den_sh.at[pl.ds(zoff, rows_per_sub)],
                        den_out.at[cid, pl.ds(zoff, rows_per_sub)])

    return ek(t32, tn, src2d, dst2d, zer_acc, zer_den)


def kernel(x, edge_index, emb, W1, b1, beta2, W2, b2):
    n = x.shape[0]
    e = edge_index.shape[1]

    np_ = NW * GB * _cdiv(n, NW * GB)
    nsp = 2048 * _cdiv(n + 1, 2048)
    epw = CH * GB * _cdiv(_cdiv(e, NW), CH * GB)
    ep = NW * epw

    xi = jnp.concatenate(
        [x[:, 0], jnp.zeros((np_ - n,), I32)])
    pad_e = jnp.full((ep - e,), nsp - 1, I32)
    src = jnp.concatenate([edge_index[0], pad_e]).reshape(ep // GB, GB)
    dst = jnp.concatenate([edge_index[1], pad_e]).reshape(ep // GB, GB)

    zer_acc = jnp.zeros((nsp, D), F32)
    zer_den = jnp.zeros((nsp,), F32)
    one = jnp.ones((), F32)

    g = _encode(emb, W1, b1)
    h = _gather_rows(g, xi, np_)

    t32, tn = _prep_tables(h, one, np_)
    acc, den = _edge_pass(t32, tn, src, dst, zer_acc, zer_den, nsp, epw)
    h1, t32, tn = _fin_prep(h[:nsp], acc, den, one, beta2, nsp)

    acc, den = _edge_pass(t32, tn, src, dst, zer_acc, zer_den, nsp, epw)
    out = _fin_head(h1, acc, den, beta2, W2, b2, nsp)
    return out[:n]


# speedup vs baseline: 1.0015x; 1.0015x over previous
import functools

import jax
import jax.numpy as jnp
from jax import lax
from jax.experimental import pallas as pl
from jax.experimental.pallas import tpu as pltpu
from jax.experimental.pallas import tpu_sc as plsc

F32 = jnp.float32
I32 = jnp.int32

_SC_PARAMS = pltpu.CompilerParams(use_tc_tiling_on_sc=False,
                                  needs_layout_passes=False)

NC = 2
NS = 16
NW = NC * NS
L = 16

D = 16
GB = 64


def _cdiv(a, b):
    return (a + b - 1) // b


def _encode_body(emb_ref, w1_ref, b1_ref, g_ref):
    acc = lax.dot_general(emb_ref[...], w1_ref[...],
                          (((1,), (0,)), ((), ())),
                          preferred_element_type=F32)
    g_ref[...] = jnp.maximum(acc + b1_ref[...], 0.0)


def _encode(emb, W1, b1):
    n, dfeat = emb.shape
    blk = 2000
    return pl.pallas_call(
        _encode_body,
        grid=(n // blk,),
        in_specs=[
            pl.BlockSpec((blk, dfeat), lambda i: (i, 0)),
            pl.BlockSpec((dfeat, D), lambda i: (0, 0)),
            pl.BlockSpec((1, D), lambda i: (0, 0)),
        ],
        out_specs=pl.BlockSpec((blk, D), lambda i: (i, 0)),
        out_shape=jax.ShapeDtypeStruct((n, D), F32),
    )(emb, W1, b1.reshape(1, D))


def _agnn_out(h, acc, den, beta):
    nrm2 = jnp.sum(h * h, axis=-1, keepdims=True)
    nrm = jnp.sqrt(nrm2)
    inv = 1.0 / jnp.maximum(nrm, 1e-12)
    s = nrm2 * inv * inv
    ex_self = jnp.exp(beta * s)
    num = acc[0] + acc[1] + ex_self * h
    dfull = (den[0] + den[1])[:, None] + ex_self
    return num / jnp.maximum(dfull, 1e-16)


def _prep_body(beta_ref, h_ref, t32_ref, tn_ref):
    h = h_ref[...]
    nrm = jnp.sqrt(jnp.sum(h * h, axis=-1, keepdims=True))
    xn = h / jnp.maximum(nrm, 1e-12)
    t32_ref[...] = jnp.concatenate([beta_ref[0] * xn, h], axis=-1)
    tn_ref[...] = xn


def _prep(h, beta, np_):
    blk = 1024
    return pl.pallas_call(
        _prep_body,
        grid=(np_ // blk,),
        in_specs=[
            pl.BlockSpec(memory_space=pltpu.SMEM),
            pl.BlockSpec((blk, D), lambda i: (i, 0)),
        ],
        out_specs=[
            pl.BlockSpec((blk, 2 * D), lambda i: (i, 0)),
            pl.BlockSpec((blk, D), lambda i: (i, 0)),
        ],
        out_shape=[
            jax.ShapeDtypeStruct((np_, 2 * D), F32),
            jax.ShapeDtypeStruct((np_, D), F32),
        ],
    )(jnp.reshape(beta.astype(F32), (1,)), h)


def _fin_prep_body(betas_ref, told_ref, acc_ref, den_ref, h1_ref, t32_ref,
                   tn_ref):
    h = told_ref[:, D:]
    h1 = _agnn_out(h, acc_ref[...], den_ref[...], betas_ref[0])
    h1_ref[...] = h1
    nrm = jnp.sqrt(jnp.sum(h1 * h1, axis=-1, keepdims=True))
    xn = h1 / jnp.maximum(nrm, 1e-12)
    t32_ref[...] = jnp.concatenate([betas_ref[1] * xn, h1], axis=-1)
    tn_ref[...] = xn


def _fin_prep(told, acc, den, beta, beta_next, np_):
    blk = 1024
    betas = jnp.stack([beta.astype(F32), beta_next.astype(F32)])
    return pl.pallas_call(
        _fin_prep_body,
        grid=(np_ // blk,),
        in_specs=[
            pl.BlockSpec(memory_space=pltpu.SMEM),
            pl.BlockSpec((blk, 2 * D), lambda i: (i, 0)),
            pl.BlockSpec((2, blk, D), lambda i: (0, i, 0)),
            pl.BlockSpec((2, blk), lambda i: (0, i)),
        ],
        out_specs=[
            pl.BlockSpec((blk, D), lambda i: (i, 0)),
            pl.BlockSpec((blk, 2 * D), lambda i: (i, 0)),
            pl.BlockSpec((blk, D), lambda i: (i, 0)),
        ],
        out_shape=[
            jax.ShapeDtypeStruct((np_, D), F32),
            jax.ShapeDtypeStruct((np_, 2 * D), F32),
            jax.ShapeDtypeStruct((np_, D), F32),
        ],
    )(betas, told, acc, den)


def _fin_head_body(beta_ref, h_ref, acc_ref, den_ref, w2_ref, b2_ref,
                   out_ref):
    h2 = _agnn_out(h_ref[...], acc_ref[...], den_ref[...], beta_ref[0])
    logits = lax.dot_general(h2, w2_ref[...],
                             (((1,), (0,)), ((), ())),
                             preferred_element_type=F32) + b2_ref[...]
    m = jnp.max(logits, axis=-1, keepdims=True)
    lse = jnp.log(jnp.sum(jnp.exp(logits - m), axis=-1, keepdims=True)) + m
    out_ref[...] = logits - lse


def _fin_head(h, acc, den, beta, W2, b2, np_):
    blk = 1024
    return pl.pallas_call(
        _fin_head_body,
        grid=(np_ // blk,),
        in_specs=[
            pl.BlockSpec(memory_space=pltpu.SMEM),
            pl.BlockSpec((blk, D), lambda i: (i, 0)),
            pl.BlockSpec((2, blk, D), lambda i: (0, i, 0)),
            pl.BlockSpec((2, blk), lambda i: (0, i)),
            pl.BlockSpec((D, D), lambda i: (0, 0)),
            pl.BlockSpec((1, D), lambda i: (0, 0)),
        ],
        out_specs=pl.BlockSpec((blk, D), lambda i: (i, 0)),
        out_shape=jax.ShapeDtypeStruct((np_, D), F32),
    )(jnp.reshape(beta.astype(F32), (1,)), h, acc, den, W2,
      b2.reshape(1, D))


def _gather(table, idx, np_):
    n_rows = np_ // NW
    n_grp = n_rows // GB
    mesh = plsc.VectorSubcoreMesh(core_axis_name="c", subcore_axis_name="s")

    @functools.partial(
        pl.kernel,
        out_type=jax.ShapeDtypeStruct((np_, D), F32),
        mesh=mesh,
        compiler_params=_SC_PARAMS,
        scratch_types=[
            pltpu.VMEM((2, GB), I32),
            pltpu.VMEM((GB, D), F32),
            pltpu.VMEM((GB, D), F32),
            pltpu.SemaphoreType.DMA,
            pltpu.SemaphoreType.DMA,
        ],
    )
    def gk(table_hbm, idx_hbm, out_hbm, idx_v, rows_a, rows_b, sem_a, sem_b):
        wid = lax.axis_index("s") * NC + lax.axis_index("c")
        base = wid * n_rows
        rows = (rows_a, rows_b)
        sems = (sem_a, sem_b)

        def start(j, b):
            off = base + j * GB
            pltpu.sync_copy(idx_hbm.at[pl.ds(off, GB)], idx_v.at[b])
            pltpu.async_copy(table_hbm.at[idx_v.at[b]], rows[b], sems[b])

        start(0, 0)

        @pl.loop(0, n_grp)
        def _grp(i):
            for b in range(2):
                @pl.when(i % 2 == b)
                def _():
                    jn = i + 1

                    @pl.when(jn < n_grp)
                    def _():
                        start(jn, 1 - b)

                    pltpu.make_async_copy(table_hbm.at[idx_v.at[b]],
                                          rows[b], sems[b]).wait()
                    pltpu.sync_copy(rows[b],
                                    out_hbm.at[pl.ds(base + i * GB, GB)])

    return gk(table, idx)


CH = 16
NB = 4


def _edge_pass(t32, tn, src2d, dst2d, zer_acc, zer_den, nsp, epw):
    n_grp = epw // GB
    rows_per_sub = nsp // NS
    mesh = plsc.VectorSubcoreMesh(core_axis_name="c", subcore_axis_name="s")

    @functools.partial(
        pl.kernel,
        out_type=[
            jax.ShapeDtypeStruct((NC, nsp, D), F32),
            jax.ShapeDtypeStruct((NC, nsp), F32),
        ],
        mesh=mesh,
        compiler_params=_SC_PARAMS,
        scratch_types=(
            [pltpu.VMEM((2 * CH, GB), I32)] * 2
            + [pltpu.VMEM((GB, 2 * D), F32)] * NB
            + [pltpu.VMEM((GB, D), F32)] * NB
            + [pltpu.VMEM((GB, D), F32)] * NB
            + [pltpu.VMEM((GB,), F32)] * NB
            + [pltpu.VMEM_SHARED((nsp, D), F32),
               pltpu.VMEM_SHARED((nsp,), F32)]
            + [pltpu.SemaphoreType.DMA] * (2 * NB)
        ),
    )
    def ek(t32_hbm, tn_hbm, src_hbm, dst_hbm, za_hbm, zd_hbm,
           acc_out, den_out, sidx, didx, *rest):
        srows = rest[0:NB]
        drows = rest[NB:2 * NB]
        pbuf = rest[2 * NB:3 * NB]
        exbuf = rest[3 * NB:4 * NB]
        acc_sh = rest[4 * NB]
        den_sh = rest[4 * NB + 1]
        semg = rest[4 * NB + 2:5 * NB + 2]
        sems = rest[5 * NB + 2:6 * NB + 2]
        cid = lax.axis_index("c")
        sid = lax.axis_index("s")
        wid = sid * NC + cid

        zoff = sid * rows_per_sub
        pltpu.sync_copy(za_hbm.at[pl.ds(zoff, rows_per_sub)],
                        acc_sh.at[pl.ds(zoff, rows_per_sub)])
        pltpu.sync_copy(zd_hbm.at[pl.ds(zoff, rows_per_sub)],
                        den_sh.at[pl.ds(zoff, rows_per_sub)])
        plsc.subcore_barrier()

        gbase = wid * n_grp
        lanes = jnp.arange(L, dtype=I32)

        def stage(c):
            par = (c % 2) * CH
            pltpu.sync_copy(src_hbm.at[pl.ds(gbase + c * CH, CH)],
                            sidx.at[pl.ds(par, CH)])
            pltpu.sync_copy(dst_hbm.at[pl.ds(gbase + c * CH, CH)],
                            didx.at[pl.ds(par, CH)])

        def idxrow(j):
            return (j % CH) + ((j // CH) % 2) * CH

        def start_gather(b, j):
            row = idxrow(j)
            pltpu.async_copy(t32_hbm.at[sidx.at[row]], srows[b], semg[b])
            pltpu.async_copy(tn_hbm.at[didx.at[row]], drows[b], semg[b])

        def wait_gather(b):
            pltpu.make_async_copy(t32_hbm.at[sidx.at[0]], srows[b],
                                  semg[b]).wait()
            pltpu.make_async_copy(tn_hbm.at[didx.at[0]], drows[b],
                                  semg[b]).wait()

        def start_scatter(b, j):
            row = idxrow(j)
            pltpu.async_copy(pbuf[b], acc_sh.at[didx.at[row]], sems[b],
                             add=True)
            pltpu.async_copy(exbuf[b], den_sh.at[didx.at[row]], sems[b],
                             add=True)

        def wait_scatter(b):
            pltpu.make_async_copy(pbuf[b], acc_sh.at[didx.at[0]],
                                  sems[b]).wait()
            pltpu.make_async_copy(exbuf[b], den_sh.at[didx.at[0]],
                                  sems[b]).wait()

        def compute(b):
            @pl.loop(0, GB // L)
            def _blk(eb):
                eidx = eb * L + lanes
                acc = jnp.zeros((L,), F32)
                for f in range(D):
                    fs = jnp.full((L,), f, I32)
                    a = plsc.load_gather(srows[b], [eidx, fs])
                    bb = plsc.load_gather(drows[b], [eidx, fs])
                    acc = acc + a * bb
                ex = jnp.exp(acc)
                plsc.store_scatter(exbuf[b], [eidx], ex)
                for f in range(D):
                    fs = jnp.full((L,), f, I32)
                    xv = plsc.load_gather(srows[b],
                                          [eidx, jnp.full((L,), D + f, I32)])
                    plsc.store_scatter(pbuf[b], [eidx, fs], ex * xv)

        stage(0)
        for b in range(NB):
            start_gather(b, b)

        @pl.loop(0, n_grp // NB)
        def _round(q):
            for b in range(NB):
                j = q * NB + b
                wait_gather(b)

                @pl.when(q > 0)
                def _():
                    wait_scatter(b)

                compute(b)
                start_scatter(b, j)
                jn = j + NB

                @pl.when(jn < n_grp)
                def _():
                    @pl.when(jn % CH == 0)
                    def _():
                        stage(jn // CH)

                    start_gather(b, jn)

        for b in range(NB):
            wait_scatter(b)
        plsc.subcore_barrier()
        pltpu.sync_copy(acc_sh.at[pl.ds(zoff, rows_per_sub)],
                        acc_out.at[cid, pl.ds(zoff, rows_per_sub)])
        pltpu.sync_copy(den_sh.at[pl.ds(zoff, rows_per_sub)],
                        den_out.at[cid, pl.ds(zoff, rows_per_sub)])

    return ek(t32, tn, src2d, dst2d, zer_acc, zer_den)


def kernel(x, edge_index, emb, W1, b1, beta2, W2, b2):
    n = x.shape[0]
    e = edge_index.shape[1]

    np_ = NW * GB * _cdiv(n, NW * GB)
    nsp = 2048 * _cdiv(n + 1, 2048)
    epw = CH * GB * _cdiv(_cdiv(e, NW), CH * GB)
    ep = NW * epw

    xi = jnp.concatenate(
        [x[:, 0], jnp.zeros((np_ - n,), I32)])
    pad_e = jnp.full((ep - e,), nsp - 1, I32)
    src = jnp.concatenate([edge_index[0], pad_e]).reshape(ep // GB, GB)
    dst = jnp.concatenate([edge_index[1], pad_e]).reshape(ep // GB, GB)

    zer_acc = jnp.zeros((nsp, D), F32)
    zer_den = jnp.zeros((nsp,), F32)
    one = jnp.ones((), F32)

    g = _encode(emb, W1, b1)
    gh = _gather(g, xi, np_)
    t32, tn = _prep(gh, one, np_)

    acc, den = _edge_pass(t32, tn, src, dst, zer_acc, zer_den, nsp, epw)
    h1, t32, tn = _fin_prep(t32[:nsp], acc, den, one, beta2, nsp)

    acc, den = _edge_pass(t32, tn, src, dst, zer_acc, zer_den, nsp, epw)
    out = _fin_head(h1, acc, den, beta2, W2, b2, nsp)
    return out[:n]


# speedup vs baseline: 1.0100x; 1.0085x over previous
import functools

import jax
import jax.numpy as jnp
from jax import lax
from jax.experimental import pallas as pl
from jax.experimental.pallas import tpu as pltpu
from jax.experimental.pallas import tpu_sc as plsc

F32 = jnp.float32
I32 = jnp.int32

_SC_PARAMS = pltpu.CompilerParams(use_tc_tiling_on_sc=False,
                                  needs_layout_passes=False)

NC = 2
NS = 16
NW = NC * NS
L = 16

D = 16
GB = 128


def _cdiv(a, b):
    return (a + b - 1) // b


def _encode_body(emb_ref, w1_ref, b1_ref, g_ref):
    acc = lax.dot_general(emb_ref[...], w1_ref[...],
                          (((1,), (0,)), ((), ())),
                          preferred_element_type=F32)
    g_ref[...] = jnp.maximum(acc + b1_ref[...], 0.0)


def _encode(emb, W1, b1):
    n, dfeat = emb.shape
    blk = 2000
    return pl.pallas_call(
        _encode_body,
        grid=(n // blk,),
        in_specs=[
            pl.BlockSpec((blk, dfeat), lambda i: (i, 0)),
            pl.BlockSpec((dfeat, D), lambda i: (0, 0)),
            pl.BlockSpec((1, D), lambda i: (0, 0)),
        ],
        out_specs=pl.BlockSpec((blk, D), lambda i: (i, 0)),
        out_shape=jax.ShapeDtypeStruct((n, D), F32),
    )(emb, W1, b1.reshape(1, D))


def _agnn_out(h, acc, den, beta):
    nrm2 = jnp.sum(h * h, axis=-1, keepdims=True)
    nrm = jnp.sqrt(nrm2)
    inv = 1.0 / jnp.maximum(nrm, 1e-12)
    s = nrm2 * inv * inv
    ex_self = jnp.exp(beta * s)
    num = acc[0] + acc[1] + ex_self * h
    dfull = (den[0] + den[1])[:, None] + ex_self
    return num / jnp.maximum(dfull, 1e-16)


def _prep_body(beta_ref, h_ref, t32_ref, tn_ref):
    h = h_ref[...]
    nrm = jnp.sqrt(jnp.sum(h * h, axis=-1, keepdims=True))
    xn = h / jnp.maximum(nrm, 1e-12)
    t32_ref[...] = jnp.concatenate([beta_ref[0] * xn, h], axis=-1)
    tn_ref[...] = xn


def _prep(h, beta, np_):
    blk = 1024
    return pl.pallas_call(
        _prep_body,
        grid=(np_ // blk,),
        in_specs=[
            pl.BlockSpec(memory_space=pltpu.SMEM),
            pl.BlockSpec((blk, D), lambda i: (i, 0)),
        ],
        out_specs=[
            pl.BlockSpec((blk, 2 * D), lambda i: (i, 0)),
            pl.BlockSpec((blk, D), lambda i: (i, 0)),
        ],
        out_shape=[
            jax.ShapeDtypeStruct((np_, 2 * D), F32),
            jax.ShapeDtypeStruct((np_, D), F32),
        ],
    )(jnp.reshape(beta.astype(F32), (1,)), h)


def _fin_prep_body(betas_ref, told_ref, acc_ref, den_ref, h1_ref, t32_ref,
                   tn_ref):
    h = told_ref[:, D:]
    h1 = _agnn_out(h, acc_ref[...], den_ref[...], betas_ref[0])
    h1_ref[...] = h1
    nrm = jnp.sqrt(jnp.sum(h1 * h1, axis=-1, keepdims=True))
    xn = h1 / jnp.maximum(nrm, 1e-12)
    t32_ref[...] = jnp.concatenate([betas_ref[1] * xn, h1], axis=-1)
    tn_ref[...] = xn


def _fin_prep(told, acc, den, beta, beta_next, np_):
    blk = 1024
    betas = jnp.stack([beta.astype(F32), beta_next.astype(F32)])
    return pl.pallas_call(
        _fin_prep_body,
        grid=(np_ // blk,),
        in_specs=[
            pl.BlockSpec(memory_space=pltpu.SMEM),
            pl.BlockSpec((blk, 2 * D), lambda i: (i, 0)),
            pl.BlockSpec((2, blk, D), lambda i: (0, i, 0)),
            pl.BlockSpec((2, blk), lambda i: (0, i)),
        ],
        out_specs=[
            pl.BlockSpec((blk, D), lambda i: (i, 0)),
            pl.BlockSpec((blk, 2 * D), lambda i: (i, 0)),
            pl.BlockSpec((blk, D), lambda i: (i, 0)),
        ],
        out_shape=[
            jax.ShapeDtypeStruct((np_, D), F32),
            jax.ShapeDtypeStruct((np_, 2 * D), F32),
            jax.ShapeDtypeStruct((np_, D), F32),
        ],
    )(betas, told, acc, den)


def _fin_head_body(beta_ref, h_ref, acc_ref, den_ref, w2_ref, b2_ref,
                   out_ref):
    h2 = _agnn_out(h_ref[...], acc_ref[...], den_ref[...], beta_ref[0])
    logits = lax.dot_general(h2, w2_ref[...],
                             (((1,), (0,)), ((), ())),
                             preferred_element_type=F32) + b2_ref[...]
    m = jnp.max(logits, axis=-1, keepdims=True)
    lse = jnp.log(jnp.sum(jnp.exp(logits - m), axis=-1, keepdims=True)) + m
    out_ref[...] = logits - lse


def _fin_head(h, acc, den, beta, W2, b2, np_):
    blk = 1024
    return pl.pallas_call(
        _fin_head_body,
        grid=(np_ // blk,),
        in_specs=[
            pl.BlockSpec(memory_space=pltpu.SMEM),
            pl.BlockSpec((blk, D), lambda i: (i, 0)),
            pl.BlockSpec((2, blk, D), lambda i: (0, i, 0)),
            pl.BlockSpec((2, blk), lambda i: (0, i)),
            pl.BlockSpec((D, D), lambda i: (0, 0)),
            pl.BlockSpec((1, D), lambda i: (0, 0)),
        ],
        out_specs=pl.BlockSpec((blk, D), lambda i: (i, 0)),
        out_shape=jax.ShapeDtypeStruct((np_, D), F32),
    )(jnp.reshape(beta.astype(F32), (1,)), h, acc, den, W2,
      b2.reshape(1, D))


def _gather(table, idx, np_):
    n_rows = np_ // NW
    n_grp = n_rows // GB
    mesh = plsc.VectorSubcoreMesh(core_axis_name="c", subcore_axis_name="s")

    @functools.partial(
        pl.kernel,
        out_type=jax.ShapeDtypeStruct((np_, D), F32),
        mesh=mesh,
        compiler_params=_SC_PARAMS,
        scratch_types=[
            pltpu.VMEM((2, GB), I32),
            pltpu.VMEM((GB, D), F32),
            pltpu.VMEM((GB, D), F32),
            pltpu.SemaphoreType.DMA,
            pltpu.SemaphoreType.DMA,
        ],
    )
    def gk(table_hbm, idx_hbm, out_hbm, idx_v, rows_a, rows_b, sem_a, sem_b):
        wid = lax.axis_index("s") * NC + lax.axis_index("c")
        base = wid * n_rows
        rows = (rows_a, rows_b)
        sems = (sem_a, sem_b)

        def start(j, b):
            off = base + j * GB
            pltpu.sync_copy(idx_hbm.at[pl.ds(off, GB)], idx_v.at[b])
            pltpu.async_copy(table_hbm.at[idx_v.at[b]], rows[b], sems[b])

        start(0, 0)

        @pl.loop(0, n_grp)
        def _grp(i):
            for b in range(2):
                @pl.when(i % 2 == b)
                def _():
                    jn = i + 1

                    @pl.when(jn < n_grp)
                    def _():
                        start(jn, 1 - b)

                    pltpu.make_async_copy(table_hbm.at[idx_v.at[b]],
                                          rows[b], sems[b]).wait()
                    pltpu.sync_copy(rows[b],
                                    out_hbm.at[pl.ds(base + i * GB, GB)])

    return gk(table, idx)


CH = 8
NB = 2


def _edge_pass(t32, tn, src2d, dst2d, zer_acc, zer_den, nsp, epw):
    n_grp = epw // GB
    rows_per_sub = nsp // NS
    mesh = plsc.VectorSubcoreMesh(core_axis_name="c", subcore_axis_name="s")

    @functools.partial(
        pl.kernel,
        out_type=[
            jax.ShapeDtypeStruct((NC, nsp, D), F32),
            jax.ShapeDtypeStruct((NC, nsp), F32),
        ],
        mesh=mesh,
        compiler_params=_SC_PARAMS,
        scratch_types=(
            [pltpu.VMEM((2 * CH, GB), I32)] * 2
            + [pltpu.VMEM((GB, 2 * D), F32)] * NB
            + [pltpu.VMEM((GB, D), F32)] * NB
            + [pltpu.VMEM((GB, D), F32)] * NB
            + [pltpu.VMEM((GB,), F32)] * NB
            + [pltpu.VMEM_SHARED((nsp, D), F32),
               pltpu.VMEM_SHARED((nsp,), F32)]
            + [pltpu.SemaphoreType.DMA] * (2 * NB)
        ),
    )
    def ek(t32_hbm, tn_hbm, src_hbm, dst_hbm, za_hbm, zd_hbm,
           acc_out, den_out, sidx, didx, *rest):
        srows = rest[0:NB]
        drows = rest[NB:2 * NB]
        pbuf = rest[2 * NB:3 * NB]
        exbuf = rest[3 * NB:4 * NB]
        acc_sh = rest[4 * NB]
        den_sh = rest[4 * NB + 1]
        semg = rest[4 * NB + 2:5 * NB + 2]
        sems = rest[5 * NB + 2:6 * NB + 2]
        cid = lax.axis_index("c")
        sid = lax.axis_index("s")
        wid = sid * NC + cid

        zoff = sid * rows_per_sub
        pltpu.sync_copy(za_hbm.at[pl.ds(zoff, rows_per_sub)],
                        acc_sh.at[pl.ds(zoff, rows_per_sub)])
        pltpu.sync_copy(zd_hbm.at[pl.ds(zoff, rows_per_sub)],
                        den_sh.at[pl.ds(zoff, rows_per_sub)])
        plsc.subcore_barrier()

        gbase = wid * n_grp
        lanes = jnp.arange(L, dtype=I32)

        def stage(c):
            par = (c % 2) * CH
            pltpu.sync_copy(src_hbm.at[pl.ds(gbase + c * CH, CH)],
                            sidx.at[pl.ds(par, CH)])
            pltpu.sync_copy(dst_hbm.at[pl.ds(gbase + c * CH, CH)],
                            didx.at[pl.ds(par, CH)])

        def idxrow(j):
            return (j % CH) + ((j // CH) % 2) * CH

        def start_gather(b, j):
            row = idxrow(j)
            pltpu.async_copy(t32_hbm.at[sidx.at[row]], srows[b], semg[b])
            pltpu.async_copy(tn_hbm.at[didx.at[row]], drows[b], semg[b])

        def wait_gather(b):
            pltpu.make_async_copy(t32_hbm.at[sidx.at[0]], srows[b],
                                  semg[b]).wait()
            pltpu.make_async_copy(tn_hbm.at[didx.at[0]], drows[b],
                                  semg[b]).wait()

        def start_scatter(b, j):
            row = idxrow(j)
            pltpu.async_copy(pbuf[b], acc_sh.at[didx.at[row]], sems[b],
                             add=True)
            pltpu.async_copy(exbuf[b], den_sh.at[didx.at[row]], sems[b],
                             add=True)

        def wait_scatter(b):
            pltpu.make_async_copy(pbuf[b], acc_sh.at[didx.at[0]],
                                  sems[b]).wait()
            pltpu.make_async_copy(exbuf[b], den_sh.at[didx.at[0]],
                                  sems[b]).wait()

        def compute(b):
            @pl.loop(0, GB // L)
            def _blk(eb):
                eidx = eb * L + lanes
                acc = jnp.zeros((L,), F32)
                for f in range(D):
                    fs = jnp.full((L,), f, I32)
                    a = plsc.load_gather(srows[b], [eidx, fs])
                    bb = plsc.load_gather(drows[b], [eidx, fs])
                    acc = acc + a * bb
                ex = jnp.exp(acc)
                plsc.store_scatter(exbuf[b], [eidx], ex)
                for f in range(D):
                    fs = jnp.full((L,), f, I32)
                    xv = plsc.load_gather(srows[b],
                                          [eidx, jnp.full((L,), D + f, I32)])
                    plsc.store_scatter(pbuf[b], [eidx, fs], ex * xv)

        stage(0)
        for b in range(NB):
            start_gather(b, b)

        @pl.loop(0, n_grp // NB)
        def _round(q):
            for b in range(NB):
                j = q * NB + b
                wait_gather(b)

                @pl.when(q > 0)
                def _():
                    wait_scatter(b)

                compute(b)
                start_scatter(b, j)
                jn = j + NB

                @pl.when(jn < n_grp)
                def _():
                    @pl.when(jn % CH == 0)
                    def _():
                        stage(jn // CH)

                    start_gather(b, jn)

        for b in range(NB):
            wait_scatter(b)
        plsc.subcore_barrier()
        pltpu.sync_copy(acc_sh.at[pl.ds(zoff, rows_per_sub)],
                        acc_out.at[cid, pl.ds(zoff, rows_per_sub)])
        pltpu.sync_copy(den_sh.at[pl.ds(zoff, rows_per_sub)],
                        den_out.at[cid, pl.ds(zoff, rows_per_sub)])

    return ek(t32, tn, src2d, dst2d, zer_acc, zer_den)


def kernel(x, edge_index, emb, W1, b1, beta2, W2, b2):
    n = x.shape[0]
    e = edge_index.shape[1]

    np_ = NW * GB * _cdiv(n, NW * GB)
    nsp = 2048 * _cdiv(n + 1, 2048)
    epw = CH * GB * _cdiv(_cdiv(e, NW), CH * GB)
    ep = NW * epw

    xi = jnp.concatenate(
        [x[:, 0], jnp.zeros((np_ - n,), I32)])
    pad_e = jnp.full((ep - e,), nsp - 1, I32)
    src = jnp.concatenate([edge_index[0], pad_e]).reshape(ep // GB, GB)
    dst = jnp.concatenate([edge_index[1], pad_e]).reshape(ep // GB, GB)

    zer_acc = jnp.zeros((nsp, D), F32)
    zer_den = jnp.zeros((nsp,), F32)
    one = jnp.ones((), F32)

    g = _encode(emb, W1, b1)
    gh = _gather(g, xi, np_)
    t32, tn = _prep(gh, one, np_)

    acc, den = _edge_pass(t32, tn, src, dst, zer_acc, zer_den, nsp, epw)
    h1, t32, tn = _fin_prep(t32[:nsp], acc, den, one, beta2, nsp)

    acc, den = _edge_pass(t32, tn, src, dst, zer_acc, zer_den, nsp, epw)
    out = _fin_head(h1, acc, den, beta2, W2, b2, nsp)
    return out[:n]
